# Initial kernel scaffold; baseline (speedup 1.0000x reference)
import jax
import jax.numpy as jnp
from jax.experimental import pallas as pl


def kernel(bbox_regression, cls_logits, anchors):
    raise NotImplementedError("write your pallas kernel here")


# speedup vs baseline: 30.9726x; 30.9726x over previous
import math

import jax
import jax.numpy as jnp
from jax import lax
from jax.experimental import pallas as pl

_N = 5000
_C = 21
_FG = 20
_K = 400
_DET = 200
_M = _FG * _K
_SCORE_THRESH = 0.01
_NMS_THRESH = 0.45
_IMG_W = 320.0
_IMG_H = 320.0
_CLIP = math.log(1000.0 / 16)
_NEG = -1e30
_CH1 = 1000
_CH2 = 200


def _prep_kernel(breg_ref, logits_ref, anchors_ref, fgc_ref, fgr_ref, boxes_ref):
    logits = logits_ref[...]
    m = jnp.max(logits, axis=1, keepdims=True)
    e = jnp.exp(logits - m)
    p = e / jnp.sum(e, axis=1, keepdims=True)
    fg = p[:, 1:]
    fg = jnp.where(fg > _SCORE_THRESH, fg, _NEG)
    fgc_ref[...] = fg
    fgr_ref[...] = jnp.transpose(fg).reshape(_FG, 1, _N)

    a = anchors_ref[...]
    widths = a[:, 2:3] - a[:, 0:1]
    heights = a[:, 3:4] - a[:, 1:2]
    ctr_x = a[:, 0:1] + 0.5 * widths
    ctr_y = a[:, 1:2] + 0.5 * heights
    r = breg_ref[...]
    dx = r[:, 0:1] / 10.0
    dy = r[:, 1:2] / 10.0
    dw = jnp.minimum(r[:, 2:3] / 5.0, _CLIP)
    dh = jnp.minimum(r[:, 3:4] / 5.0, _CLIP)
    pcx = dx * widths + ctr_x
    pcy = dy * heights + ctr_y
    pw = jnp.exp(dw) * widths
    ph = jnp.exp(dh) * heights
    x1 = jnp.clip(pcx - 0.5 * pw, 0.0, _IMG_W)
    y1 = jnp.clip(pcy - 0.5 * ph, 0.0, _IMG_H)
    x2 = jnp.clip(pcx + 0.5 * pw, 0.0, _IMG_W)
    y2 = jnp.clip(pcy + 0.5 * ph, 0.0, _IMG_H)
    boxes_ref[...] = jnp.concatenate([x1, y1, x2, y2], axis=1)


def _class_kernel(fgc_ref, fgr_ref, boxes_ref, sc_ref, bx_ref, keep_ref):
    c = pl.program_id(0)
    onehot_c = (lax.broadcasted_iota(jnp.int32, (_FG, 1), 0) == c).astype(jnp.float32)
    s_row = fgr_ref[0]
    iidx = lax.broadcasted_iota(jnp.int32, (1, _N), 1)

    def rank_chunk(j, rank):
        s_c = lax.dot_general(
            fgc_ref[pl.ds(j * _CH1, _CH1), :], onehot_c,
            (((1,), (0,)), ((), ())), preferred_element_type=jnp.float32, precision=lax.Precision.HIGHEST)
        jidx = j * _CH1 + lax.broadcasted_iota(jnp.int32, (_CH1, 1), 0)
        cmp = (s_c > s_row) | ((s_c == s_row) & (jidx < iidx))
        return rank + jnp.sum(cmp.astype(jnp.float32), axis=0, keepdims=True)

    rank = lax.fori_loop(0, _N // _CH1, rank_chunk,
                         jnp.zeros((1, _N), jnp.float32))

    r_io = lax.broadcasted_iota(jnp.int32, (_K, _N), 0)
    sel = (r_io == rank.astype(jnp.int32)).astype(jnp.float32)
    boxes = boxes_ref[...]
    sc_top = lax.dot_general(s_row, sel, (((1,), (1,)), ((), ())),
                             preferred_element_type=jnp.float32, precision=lax.Precision.HIGHEST)
    bx_top = lax.dot_general(sel, boxes, (((1,), (0,)), ((), ())),
                             preferred_element_type=jnp.float32, precision=lax.Precision.HIGHEST)
    rows4 = lax.dot_general(boxes, sel, (((0,), (1,)), ((), ())),
                            preferred_element_type=jnp.float32, precision=lax.Precision.HIGHEST)

    x1c, y1c, x2c, y2c = (bx_top[:, 0:1], bx_top[:, 1:2],
                          bx_top[:, 2:3], bx_top[:, 3:4])
    x1r, y1r, x2r, y2r = (rows4[0:1, :], rows4[1:2, :],
                          rows4[2:3, :], rows4[3:4, :])
    area_c = (x2c - x1c) * (y2c - y1c)
    area_r = (x2r - x1r) * (y2r - y1r)
    xx1 = jnp.maximum(x1c, x1r)
    yy1 = jnp.maximum(y1c, y1r)
    xx2 = jnp.minimum(x2c, x2r)
    yy2 = jnp.minimum(y2c, y2r)
    inter = jnp.maximum(xx2 - xx1, 0.0) * jnp.maximum(yy2 - yy1, 0.0)
    union = jnp.maximum(area_c + area_r - inter, 1e-9)
    iou = inter / union
    ri = lax.broadcasted_iota(jnp.int32, (_K, _K), 0)
    ci = lax.broadcasted_iota(jnp.int32, (_K, _K), 1)
    sup = ((iou > _NMS_THRESH) & (ri < ci)).astype(jnp.float32)

    valid_f = (sc_top > -1e29).astype(jnp.float32)

    def cond_fn(carry):
        return carry[1]

    def body_fn(carry):
        k = carry[0]
        s = lax.dot_general(k, sup, (((1,), (0,)), ((), ())),
                            preferred_element_type=jnp.float32, precision=lax.Precision.HIGHEST)
        kn = valid_f * (s < 0.5).astype(jnp.float32)
        return (kn, jnp.any(kn != k))

    keep_f, _ = lax.while_loop(cond_fn, body_fn, (valid_f, True))

    sc_ref[0] = sc_top
    bx_ref[0] = bx_top
    keep_ref[0] = keep_f


def _final_kernel(s_row_ref, s_col_ref, k_row_ref, k_col_ref, bx_ref,
                  boxes_out, scores_out, labels_out):
    s_row = s_row_ref[...]
    k_row = k_row_ref[...]
    a_row = jnp.where(k_row > 0.5, s_row, _NEG)
    iidx = lax.broadcasted_iota(jnp.int32, (1, _M), 1)

    def rank_chunk(j, rank):
        s_c = s_col_ref[pl.ds(j * _CH2, _CH2), :]
        k_c = k_col_ref[pl.ds(j * _CH2, _CH2), :]
        a_c = jnp.where(k_c > 0.5, s_c, _NEG)
        jidx = j * _CH2 + lax.broadcasted_iota(jnp.int32, (_CH2, 1), 0)
        cmp = (a_c > a_row) | ((a_c == a_row) &
                               ((s_c > s_row) | ((s_c == s_row) & (jidx < iidx))))
        return rank + jnp.sum(cmp.astype(jnp.float32), axis=0, keepdims=True)

    rank = lax.fori_loop(0, _M // _CH2, rank_chunk,
                         jnp.zeros((1, _M), jnp.float32))

    r_io = lax.broadcasted_iota(jnp.int32, (_DET, _M), 0)
    sel = (r_io == rank.astype(jnp.int32)).astype(jnp.float32)
    boxes_out[...] = lax.dot_general(sel, bx_ref[...], (((1,), (0,)), ((), ())),
                                     preferred_element_type=jnp.float32, precision=lax.Precision.HIGHEST)
    ssel = lax.dot_general(s_row, sel, (((1,), (1,)), ((), ())),
                           preferred_element_type=jnp.float32, precision=lax.Precision.HIGHEST)
    scores_out[...] = jnp.where(ssel <= -1e29, -jnp.inf, ssel)
    lab_row = (iidx // _K + 1).astype(jnp.float32)
    lsel = lax.dot_general(lab_row, sel, (((1,), (1,)), ((), ())),
                           preferred_element_type=jnp.float32, precision=lax.Precision.HIGHEST)
    labels_out[...] = (lsel + 0.5).astype(jnp.int32)


def kernel(bbox_regression, cls_logits, anchors):
    f32 = jnp.float32
    fgc, fgr, boxes = pl.pallas_call(
        _prep_kernel,
        out_shape=[
            jax.ShapeDtypeStruct((_N, _FG), f32),
            jax.ShapeDtypeStruct((_FG, 1, _N), f32),
            jax.ShapeDtypeStruct((_N, 4), f32),
        ],
    )(bbox_regression, cls_logits, anchors)

    sc_top, bx_top, keep = pl.pallas_call(
        _class_kernel,
        grid=(_FG,),
        in_specs=[
            pl.BlockSpec((_N, _FG), lambda c: (0, 0)),
            pl.BlockSpec((1, 1, _N), lambda c: (c, 0, 0)),
            pl.BlockSpec((_N, 4), lambda c: (0, 0)),
        ],
        out_specs=[
            pl.BlockSpec((1, 1, _K), lambda c: (c, 0, 0)),
            pl.BlockSpec((1, _K, 4), lambda c: (c, 0, 0)),
            pl.BlockSpec((1, 1, _K), lambda c: (c, 0, 0)),
        ],
        out_shape=[
            jax.ShapeDtypeStruct((_FG, 1, _K), f32),
            jax.ShapeDtypeStruct((_FG, _K, 4), f32),
            jax.ShapeDtypeStruct((_FG, 1, _K), f32),
        ],
    )(fgc, fgr, boxes)

    s_row = sc_top.reshape(1, _M)
    s_col = sc_top.reshape(_M, 1)
    k_row = keep.reshape(1, _M)
    k_col = keep.reshape(_M, 1)
    bx = bx_top.reshape(_M, 4)

    boxes_o, scores_o, labels_o = pl.pallas_call(
        _final_kernel,
        out_shape=[
            jax.ShapeDtypeStruct((_DET, 4), f32),
            jax.ShapeDtypeStruct((1, _DET), f32),
            jax.ShapeDtypeStruct((1, _DET), jnp.int32),
        ],
    )(s_row, s_col, k_row, k_col, bx)

    return boxes_o, scores_o.reshape(_DET), labels_o.reshape(_DET)


# speedup vs baseline: 31.1632x; 1.0062x over previous
import math

import jax
import jax.numpy as jnp
from jax import lax
from jax.experimental import pallas as pl

_N = 5000
_C = 21
_FG = 20
_K = 400
_DET = 200
_M = _FG * _K
_SCORE_THRESH = 0.01
_NMS_THRESH = 0.45
_IMG_W = 320.0
_IMG_H = 320.0
_CLIP = math.log(1000.0 / 16)
_NEG = -1e30
_CH1 = 1000
_CH2 = 200


def _prep_kernel(breg_ref, logits_ref, anchors_ref, fgc_ref, fgr_ref, boxes_ref):
    logits = logits_ref[...]
    m = jnp.max(logits, axis=1, keepdims=True)
    e = jnp.exp(logits - m)
    p = e / jnp.sum(e, axis=1, keepdims=True)
    fg = p[:, 1:]
    fg = jnp.where(fg > _SCORE_THRESH, fg, _NEG)
    fgc_ref[...] = fg
    fgr_ref[...] = jnp.transpose(fg).reshape(_FG, 1, _N)

    a = anchors_ref[...]
    widths = a[:, 2:3] - a[:, 0:1]
    heights = a[:, 3:4] - a[:, 1:2]
    ctr_x = a[:, 0:1] + 0.5 * widths
    ctr_y = a[:, 1:2] + 0.5 * heights
    r = breg_ref[...]
    dx = r[:, 0:1] / 10.0
    dy = r[:, 1:2] / 10.0
    dw = jnp.minimum(r[:, 2:3] / 5.0, _CLIP)
    dh = jnp.minimum(r[:, 3:4] / 5.0, _CLIP)
    pcx = dx * widths + ctr_x
    pcy = dy * heights + ctr_y
    pw = jnp.exp(dw) * widths
    ph = jnp.exp(dh) * heights
    x1 = jnp.clip(pcx - 0.5 * pw, 0.0, _IMG_W)
    y1 = jnp.clip(pcy - 0.5 * ph, 0.0, _IMG_H)
    x2 = jnp.clip(pcx + 0.5 * pw, 0.0, _IMG_W)
    y2 = jnp.clip(pcy + 0.5 * ph, 0.0, _IMG_H)
    boxes_ref[...] = jnp.concatenate([x1, y1, x2, y2], axis=1)


def _class_kernel(fgc_ref, fgr_ref, boxes_ref, sc_ref, bx_ref, keep_ref):
    c = pl.program_id(0)
    onehot_c = (lax.broadcasted_iota(jnp.int32, (_FG, 1), 0) == c).astype(jnp.float32)
    s_row = fgr_ref[0]

    sub_l = lax.broadcasted_iota(jnp.int32, (_CH1, 1), 0)
    lane_l = lax.broadcasted_iota(jnp.int32, (1, _CH1), 1)
    diag_lt = lane_l < sub_l
    diag_gt = lane_l > sub_l
    rank_parts = []
    for a in range(_N // _CH1):
        lo, hi = a * _CH1, (a + 1) * _CH1
        s_c = lax.dot_general(
            fgc_ref[lo:hi, :], onehot_c,
            (((1,), (0,)), ((), ())), preferred_element_type=jnp.float32,
            precision=lax.Precision.HIGHEST)
        s_diag = s_row[:, lo:hi]
        acc = (lo + sub_l).astype(jnp.float32)
        acc = acc - jnp.sum(((s_c > s_diag) & diag_lt).astype(jnp.float32),
                            axis=1, keepdims=True)
        acc = acc + jnp.sum(((s_diag > s_c) & diag_gt).astype(jnp.float32),
                            axis=1, keepdims=True)
        if lo > 0:
            acc = acc - jnp.sum((s_c > s_row[:, :lo]).astype(jnp.float32),
                                axis=1, keepdims=True)
        if hi < _N:
            acc = acc + jnp.sum((s_row[:, hi:] > s_c).astype(jnp.float32),
                                axis=1, keepdims=True)
        rank_parts.append(acc)
    rank_col = jnp.concatenate(rank_parts, axis=0)

    k_io = lax.broadcasted_iota(jnp.int32, (_N, _K), 1)
    selt = (k_io == rank_col.astype(jnp.int32)).astype(jnp.float32)
    boxes = boxes_ref[...]
    sc_top = lax.dot_general(s_row, selt, (((1,), (0,)), ((), ())),
                             preferred_element_type=jnp.float32, precision=lax.Precision.HIGHEST)
    bx_top = lax.dot_general(selt, boxes, (((0,), (0,)), ((), ())),
                             preferred_element_type=jnp.float32, precision=lax.Precision.HIGHEST)
    rows4 = lax.dot_general(boxes, selt, (((0,), (0,)), ((), ())),
                            preferred_element_type=jnp.float32, precision=lax.Precision.HIGHEST)

    x1c, y1c, x2c, y2c = (bx_top[:, 0:1], bx_top[:, 1:2],
                          bx_top[:, 2:3], bx_top[:, 3:4])
    x1r, y1r, x2r, y2r = (rows4[0:1, :], rows4[1:2, :],
                          rows4[2:3, :], rows4[3:4, :])
    area_c = (x2c - x1c) * (y2c - y1c)
    area_r = (x2r - x1r) * (y2r - y1r)
    xx1 = jnp.maximum(x1c, x1r)
    yy1 = jnp.maximum(y1c, y1r)
    xx2 = jnp.minimum(x2c, x2r)
    yy2 = jnp.minimum(y2c, y2r)
    inter = jnp.maximum(xx2 - xx1, 0.0) * jnp.maximum(yy2 - yy1, 0.0)
    union = jnp.maximum(area_c + area_r - inter, 1e-9)
    iou = inter / union
    ri = lax.broadcasted_iota(jnp.int32, (_K, _K), 0)
    ci = lax.broadcasted_iota(jnp.int32, (_K, _K), 1)
    sup = ((iou > _NMS_THRESH) & (ri < ci)).astype(jnp.float32)

    valid_f = (sc_top > -1e29).astype(jnp.float32)

    def cond_fn(carry):
        return carry[1]

    def body_fn(carry):
        k = carry[0]
        s = lax.dot_general(k, sup, (((1,), (0,)), ((), ())),
                            preferred_element_type=jnp.float32, precision=lax.Precision.HIGHEST)
        kn = valid_f * (s < 0.5).astype(jnp.float32)
        return (kn, jnp.any(kn != k))

    keep_f, _ = lax.while_loop(cond_fn, body_fn, (valid_f, True))

    sc_ref[0] = sc_top
    bx_ref[0] = bx_top
    keep_ref[0] = keep_f


def _final_kernel(s_row_ref, s_col_ref, k_row_ref, k_col_ref, bx_ref,
                  boxes_out, scores_out, labels_out):
    s_row = s_row_ref[...]
    k_row = k_row_ref[...]
    a_row = jnp.where(k_row > 0.5, s_row, _NEG)
    iidx = lax.broadcasted_iota(jnp.int32, (1, _M), 1)

    def rank_chunk(j, rank):
        s_c = s_col_ref[pl.ds(j * _CH2, _CH2), :]
        k_c = k_col_ref[pl.ds(j * _CH2, _CH2), :]
        a_c = jnp.where(k_c > 0.5, s_c, _NEG)
        jidx = j * _CH2 + lax.broadcasted_iota(jnp.int32, (_CH2, 1), 0)
        cmp = (a_c > a_row) | ((a_c == a_row) &
                               ((s_c > s_row) | ((s_c == s_row) & (jidx < iidx))))
        return rank + jnp.sum(cmp.astype(jnp.float32), axis=0, keepdims=True)

    rank = lax.fori_loop(0, _M // _CH2, rank_chunk,
                         jnp.zeros((1, _M), jnp.float32))

    r_io = lax.broadcasted_iota(jnp.int32, (_DET, _M), 0)
    sel = (r_io == rank.astype(jnp.int32)).astype(jnp.float32)
    boxes_out[...] = lax.dot_general(sel, bx_ref[...], (((1,), (0,)), ((), ())),
                                     preferred_element_type=jnp.float32, precision=lax.Precision.HIGHEST)
    ssel = lax.dot_general(s_row, sel, (((1,), (1,)), ((), ())),
                           preferred_element_type=jnp.float32, precision=lax.Precision.HIGHEST)
    scores_out[...] = jnp.where(ssel <= -1e29, -jnp.inf, ssel)
    lab_row = (iidx // _K + 1).astype(jnp.float32)
    lsel = lax.dot_general(lab_row, sel, (((1,), (1,)), ((), ())),
                           preferred_element_type=jnp.float32, precision=lax.Precision.HIGHEST)
    labels_out[...] = (lsel + 0.5).astype(jnp.int32)


def kernel(bbox_regression, cls_logits, anchors):
    f32 = jnp.float32
    fgc, fgr, boxes = pl.pallas_call(
        _prep_kernel,
        out_shape=[
            jax.ShapeDtypeStruct((_N, _FG), f32),
            jax.ShapeDtypeStruct((_FG, 1, _N), f32),
            jax.ShapeDtypeStruct((_N, 4), f32),
        ],
    )(bbox_regression, cls_logits, anchors)

    sc_top, bx_top, keep = pl.pallas_call(
        _class_kernel,
        grid=(_FG,),
        in_specs=[
            pl.BlockSpec((_N, _FG), lambda c: (0, 0)),
            pl.BlockSpec((1, 1, _N), lambda c: (c, 0, 0)),
            pl.BlockSpec((_N, 4), lambda c: (0, 0)),
        ],
        out_specs=[
            pl.BlockSpec((1, 1, _K), lambda c: (c, 0, 0)),
            pl.BlockSpec((1, _K, 4), lambda c: (c, 0, 0)),
            pl.BlockSpec((1, 1, _K), lambda c: (c, 0, 0)),
        ],
        out_shape=[
            jax.ShapeDtypeStruct((_FG, 1, _K), f32),
            jax.ShapeDtypeStruct((_FG, _K, 4), f32),
            jax.ShapeDtypeStruct((_FG, 1, _K), f32),
        ],
    )(fgc, fgr, boxes)

    s_row = sc_top.reshape(1, _M)
    s_col = sc_top.reshape(_M, 1)
    k_row = keep.reshape(1, _M)
    k_col = keep.reshape(_M, 1)
    bx = bx_top.reshape(_M, 4)

    boxes_o, scores_o, labels_o = pl.pallas_call(
        _final_kernel,
        out_shape=[
            jax.ShapeDtypeStruct((_DET, 4), f32),
            jax.ShapeDtypeStruct((1, _DET), f32),
            jax.ShapeDtypeStruct((1, _DET), jnp.int32),
        ],
    )(s_row, s_col, k_row, k_col, bx)

    return boxes_o, scores_o.reshape(_DET), labels_o.reshape(_DET)


# speedup vs baseline: 35.3391x; 1.1340x over previous
import math

import jax
import jax.numpy as jnp
from jax import lax
from jax.experimental import pallas as pl

_N = 5000
_C = 21
_FG = 20
_K = 400
_DET = 200
_M = _FG * _K
_SCORE_THRESH = 0.01
_NMS_THRESH = 0.45
_IMG_W = 320.0
_IMG_H = 320.0
_CLIP = math.log(1000.0 / 16)
_NEG = -1e30
_CH1 = 1000
_CH2 = 200


def _prep_kernel(breg_ref, logits_ref, anchors_ref, fgc_ref, fgr_ref, boxes_ref):
    logits = logits_ref[...]
    m = jnp.max(logits, axis=1, keepdims=True)
    e = jnp.exp(logits - m)
    p = e / jnp.sum(e, axis=1, keepdims=True)
    fg = p[:, 1:]
    fg = jnp.where(fg > _SCORE_THRESH, fg, _NEG)
    fgc_ref[...] = fg
    fgr_ref[...] = jnp.transpose(fg).reshape(_FG, 1, _N)

    a = anchors_ref[...]
    widths = a[:, 2:3] - a[:, 0:1]
    heights = a[:, 3:4] - a[:, 1:2]
    ctr_x = a[:, 0:1] + 0.5 * widths
    ctr_y = a[:, 1:2] + 0.5 * heights
    r = breg_ref[...]
    dx = r[:, 0:1] / 10.0
    dy = r[:, 1:2] / 10.0
    dw = jnp.minimum(r[:, 2:3] / 5.0, _CLIP)
    dh = jnp.minimum(r[:, 3:4] / 5.0, _CLIP)
    pcx = dx * widths + ctr_x
    pcy = dy * heights + ctr_y
    pw = jnp.exp(dw) * widths
    ph = jnp.exp(dh) * heights
    x1 = jnp.clip(pcx - 0.5 * pw, 0.0, _IMG_W)
    y1 = jnp.clip(pcy - 0.5 * ph, 0.0, _IMG_H)
    x2 = jnp.clip(pcx + 0.5 * pw, 0.0, _IMG_W)
    y2 = jnp.clip(pcy + 0.5 * ph, 0.0, _IMG_H)
    boxes_ref[...] = jnp.concatenate([x1, y1, x2, y2], axis=1)


def _class_kernel(fgc_ref, fgr_ref, boxes_ref, sc_ref, bx_ref, keep_ref):
    c = pl.program_id(0)
    s_row = fgr_ref[0]
    lane_c = lax.broadcasted_iota(jnp.int32, (_N, _FG), 1) == c
    s_col = jnp.sum(jnp.where(lane_c, fgc_ref[...], 0.0),
                    axis=1, keepdims=True)

    sub_l = lax.broadcasted_iota(jnp.int32, (_CH1, 1), 0)
    lane_l = lax.broadcasted_iota(jnp.int32, (1, _CH1), 1)
    diag_lt = lane_l < sub_l
    diag_gt = lane_l > sub_l
    rank_parts = []
    for a in range(_N // _CH1):
        lo, hi = a * _CH1, (a + 1) * _CH1
        s_c = s_col[lo:hi, :]
        s_diag = s_row[:, lo:hi]
        acc = (lo + sub_l).astype(jnp.float32)
        acc = acc - jnp.sum(((s_c > s_diag) & diag_lt).astype(jnp.float32),
                            axis=1, keepdims=True)
        acc = acc + jnp.sum(((s_diag > s_c) & diag_gt).astype(jnp.float32),
                            axis=1, keepdims=True)
        if lo > 0:
            acc = acc - jnp.sum((s_c > s_row[:, :lo]).astype(jnp.float32),
                                axis=1, keepdims=True)
        if hi < _N:
            acc = acc + jnp.sum((s_row[:, hi:] > s_c).astype(jnp.float32),
                                axis=1, keepdims=True)
        rank_parts.append(acc)
    rank_col = jnp.concatenate(rank_parts, axis=0)

    k_io = lax.broadcasted_iota(jnp.int32, (_N, _K), 1)
    selt = (k_io == rank_col.astype(jnp.int32)).astype(jnp.float32)
    boxes = boxes_ref[...]
    sc_top = lax.dot_general(s_row, selt, (((1,), (0,)), ((), ())),
                             preferred_element_type=jnp.float32, precision=lax.Precision.HIGHEST)
    bx_top = lax.dot_general(selt, boxes, (((0,), (0,)), ((), ())),
                             preferred_element_type=jnp.float32, precision=lax.Precision.HIGHEST)
    rows4 = lax.dot_general(boxes, selt, (((0,), (0,)), ((), ())),
                            preferred_element_type=jnp.float32, precision=lax.Precision.HIGHEST)

    x1c, y1c, x2c, y2c = (bx_top[:, 0:1], bx_top[:, 1:2],
                          bx_top[:, 2:3], bx_top[:, 3:4])
    x1r, y1r, x2r, y2r = (rows4[0:1, :], rows4[1:2, :],
                          rows4[2:3, :], rows4[3:4, :])
    area_c = (x2c - x1c) * (y2c - y1c)
    area_r = (x2r - x1r) * (y2r - y1r)
    xx1 = jnp.maximum(x1c, x1r)
    yy1 = jnp.maximum(y1c, y1r)
    xx2 = jnp.minimum(x2c, x2r)
    yy2 = jnp.minimum(y2c, y2r)
    inter = jnp.maximum(xx2 - xx1, 0.0) * jnp.maximum(yy2 - yy1, 0.0)
    union = jnp.maximum(area_c + area_r - inter, 1e-9)
    iou = inter / union
    ri = lax.broadcasted_iota(jnp.int32, (_K, _K), 0)
    ci = lax.broadcasted_iota(jnp.int32, (_K, _K), 1)
    sup = ((iou > _NMS_THRESH) & (ri < ci)).astype(jnp.float32)

    valid_f = (sc_top > -1e29).astype(jnp.float32)

    def cond_fn(carry):
        return carry[1]

    def body_fn(carry):
        k = carry[0]
        s = lax.dot_general(k, sup, (((1,), (0,)), ((), ())),
                            preferred_element_type=jnp.float32)
        kn = valid_f * (s < 0.5).astype(jnp.float32)
        return (kn, jnp.any(kn != k))

    keep_f, _ = lax.while_loop(cond_fn, body_fn, (valid_f, True))

    sc_ref[0] = sc_top
    bx_ref[0] = bx_top
    keep_ref[0] = keep_f


def _final_kernel(s_row_ref, s_col_ref, k_row_ref, k_col_ref, bx_ref,
                  boxes_out, scores_out, labels_out):
    s_row = s_row_ref[...]
    k_row = k_row_ref[...]
    a_row = jnp.where(k_row > 0.5, s_row, _NEG)
    iidx = lax.broadcasted_iota(jnp.int32, (1, _M), 1)

    def rank_chunk(j, rank):
        s_c = s_col_ref[pl.ds(j * _CH2, _CH2), :]
        k_c = k_col_ref[pl.ds(j * _CH2, _CH2), :]
        a_c = jnp.where(k_c > 0.5, s_c, _NEG)
        jidx = j * _CH2 + lax.broadcasted_iota(jnp.int32, (_CH2, 1), 0)
        cmp = (a_c > a_row) | ((a_c == a_row) &
                               ((s_c > s_row) | ((s_c == s_row) & (jidx < iidx))))
        return rank + jnp.sum(cmp.astype(jnp.float32), axis=0, keepdims=True)

    rank = lax.fori_loop(0, _M // _CH2, rank_chunk,
                         jnp.zeros((1, _M), jnp.float32))

    r_io = lax.broadcasted_iota(jnp.int32, (_DET, _M), 0)
    sel = (r_io == rank.astype(jnp.int32)).astype(jnp.float32)
    boxes_out[...] = lax.dot_general(sel, bx_ref[...], (((1,), (0,)), ((), ())),
                                     preferred_element_type=jnp.float32, precision=lax.Precision.HIGHEST)
    ssel = lax.dot_general(s_row, sel, (((1,), (1,)), ((), ())),
                           preferred_element_type=jnp.float32, precision=lax.Precision.HIGHEST)
    scores_out[...] = jnp.where(ssel <= -1e29, -jnp.inf, ssel)
    lab_row = (iidx // _K + 1).astype(jnp.float32)
    lsel = lax.dot_general(lab_row, sel, (((1,), (1,)), ((), ())),
                           preferred_element_type=jnp.float32, precision=lax.Precision.HIGHEST)
    labels_out[...] = (lsel + 0.5).astype(jnp.int32)


def kernel(bbox_regression, cls_logits, anchors):
    f32 = jnp.float32
    fgc, fgr, boxes = pl.pallas_call(
        _prep_kernel,
        out_shape=[
            jax.ShapeDtypeStruct((_N, _FG), f32),
            jax.ShapeDtypeStruct((_FG, 1, _N), f32),
            jax.ShapeDtypeStruct((_N, 4), f32),
        ],
    )(bbox_regression, cls_logits, anchors)

    sc_top, bx_top, keep = pl.pallas_call(
        _class_kernel,
        grid=(_FG,),
        in_specs=[
            pl.BlockSpec((_N, _FG), lambda c: (0, 0)),
            pl.BlockSpec((1, 1, _N), lambda c: (c, 0, 0)),
            pl.BlockSpec((_N, 4), lambda c: (0, 0)),
        ],
        out_specs=[
            pl.BlockSpec((1, 1, _K), lambda c: (c, 0, 0)),
            pl.BlockSpec((1, _K, 4), lambda c: (c, 0, 0)),
            pl.BlockSpec((1, 1, _K), lambda c: (c, 0, 0)),
        ],
        out_shape=[
            jax.ShapeDtypeStruct((_FG, 1, _K), f32),
            jax.ShapeDtypeStruct((_FG, _K, 4), f32),
            jax.ShapeDtypeStruct((_FG, 1, _K), f32),
        ],
    )(fgc, fgr, boxes)

    s_row = sc_top.reshape(1, _M)
    s_col = sc_top.reshape(_M, 1)
    k_row = keep.reshape(1, _M)
    k_col = keep.reshape(_M, 1)
    bx = bx_top.reshape(_M, 4)

    boxes_o, scores_o, labels_o = pl.pallas_call(
        _final_kernel,
        out_shape=[
            jax.ShapeDtypeStruct((_DET, 4), f32),
            jax.ShapeDtypeStruct((1, _DET), f32),
            jax.ShapeDtypeStruct((1, _DET), jnp.int32),
        ],
    )(s_row, s_col, k_row, k_col, bx)

    return boxes_o, scores_o.reshape(_DET), labels_o.reshape(_DET)


# speedup vs baseline: 37.2412x; 1.0538x over previous
import math

import jax
import jax.numpy as jnp
from jax import lax
from jax.experimental import pallas as pl

_N = 5000
_C = 21
_FG = 20
_K = 400
_DET = 200
_M = _FG * _K
_SCORE_THRESH = 0.01
_NMS_THRESH = 0.45
_IMG_W = 320.0
_IMG_H = 320.0
_CLIP = math.log(1000.0 / 16)
_NEG = -1e30
_CH1 = 1000
_CH2 = 400


def _prep_kernel(breg_ref, logits_ref, anchors_ref, fgc_ref, fgr_ref, boxes_ref):
    logits = logits_ref[...]
    m = jnp.max(logits, axis=1, keepdims=True)
    e = jnp.exp(logits - m)
    p = e / jnp.sum(e, axis=1, keepdims=True)
    fg = p[:, 1:]
    fg = jnp.where(fg > _SCORE_THRESH, fg, _NEG)
    fgc_ref[...] = fg
    fgr_ref[...] = jnp.transpose(fg).reshape(_FG, 1, _N)

    a = anchors_ref[...]
    widths = a[:, 2:3] - a[:, 0:1]
    heights = a[:, 3:4] - a[:, 1:2]
    ctr_x = a[:, 0:1] + 0.5 * widths
    ctr_y = a[:, 1:2] + 0.5 * heights
    r = breg_ref[...]
    dx = r[:, 0:1] / 10.0
    dy = r[:, 1:2] / 10.0
    dw = jnp.minimum(r[:, 2:3] / 5.0, _CLIP)
    dh = jnp.minimum(r[:, 3:4] / 5.0, _CLIP)
    pcx = dx * widths + ctr_x
    pcy = dy * heights + ctr_y
    pw = jnp.exp(dw) * widths
    ph = jnp.exp(dh) * heights
    x1 = jnp.clip(pcx - 0.5 * pw, 0.0, _IMG_W)
    y1 = jnp.clip(pcy - 0.5 * ph, 0.0, _IMG_H)
    x2 = jnp.clip(pcx + 0.5 * pw, 0.0, _IMG_W)
    y2 = jnp.clip(pcy + 0.5 * ph, 0.0, _IMG_H)
    boxes_ref[...] = jnp.concatenate([x1, y1, x2, y2], axis=1)


def _class_kernel(fgc_ref, fgr_ref, boxes_ref, sc_ref, bx_ref, keep_ref):
    c = pl.program_id(0)
    s_row = fgr_ref[0]
    lane_c = lax.broadcasted_iota(jnp.int32, (_N, _FG), 1) == c
    s_col = jnp.sum(jnp.where(lane_c, fgc_ref[...], 0.0),
                    axis=1, keepdims=True)

    sub_l = lax.broadcasted_iota(jnp.int32, (_CH1, 1), 0)
    lane_l = lax.broadcasted_iota(jnp.int32, (1, _CH1), 1)
    diag_lt = lane_l < sub_l
    diag_gt = lane_l > sub_l
    rank_parts = []
    for a in range(_N // _CH1):
        lo, hi = a * _CH1, (a + 1) * _CH1
        s_c = s_col[lo:hi, :]
        s_diag = s_row[:, lo:hi]
        acc = (lo + sub_l).astype(jnp.float32)
        acc = acc - jnp.sum(((s_c > s_diag) & diag_lt).astype(jnp.float32),
                            axis=1, keepdims=True)
        acc = acc + jnp.sum(((s_diag > s_c) & diag_gt).astype(jnp.float32),
                            axis=1, keepdims=True)
        if lo > 0:
            acc = acc - jnp.sum((s_c > s_row[:, :lo]).astype(jnp.float32),
                                axis=1, keepdims=True)
        if hi < _N:
            acc = acc + jnp.sum((s_row[:, hi:] > s_c).astype(jnp.float32),
                                axis=1, keepdims=True)
        rank_parts.append(acc)
    rank_col = jnp.concatenate(rank_parts, axis=0)

    k_io = lax.broadcasted_iota(jnp.int32, (_N, _K), 1)
    selt = (k_io == rank_col.astype(jnp.int32)).astype(jnp.float32)
    boxes = boxes_ref[...]
    sc_top = lax.dot_general(s_row, selt, (((1,), (0,)), ((), ())),
                             preferred_element_type=jnp.float32, precision=lax.Precision.HIGHEST)
    bx_top = lax.dot_general(selt, boxes, (((0,), (0,)), ((), ())),
                             preferred_element_type=jnp.float32, precision=lax.Precision.HIGHEST)
    rows4 = jnp.transpose(bx_top)

    x1c, y1c, x2c, y2c = (bx_top[:, 0:1], bx_top[:, 1:2],
                          bx_top[:, 2:3], bx_top[:, 3:4])
    x1r, y1r, x2r, y2r = (rows4[0:1, :], rows4[1:2, :],
                          rows4[2:3, :], rows4[3:4, :])
    area_c = (x2c - x1c) * (y2c - y1c)
    area_r = (x2r - x1r) * (y2r - y1r)
    xx1 = jnp.maximum(x1c, x1r)
    yy1 = jnp.maximum(y1c, y1r)
    xx2 = jnp.minimum(x2c, x2r)
    yy2 = jnp.minimum(y2c, y2r)
    inter = jnp.maximum(xx2 - xx1, 0.0) * jnp.maximum(yy2 - yy1, 0.0)
    union = jnp.maximum(area_c + area_r - inter, 1e-9)
    iou = inter / union
    ri = lax.broadcasted_iota(jnp.int32, (_K, _K), 0)
    ci = lax.broadcasted_iota(jnp.int32, (_K, _K), 1)
    sup = ((iou > _NMS_THRESH) & (ri < ci)).astype(jnp.float32)

    valid_f = (sc_top > -1e29).astype(jnp.float32)

    def cond_fn(carry):
        return carry[1]

    def body_fn(carry):
        k = carry[0]
        s = lax.dot_general(k, sup, (((1,), (0,)), ((), ())),
                            preferred_element_type=jnp.float32)
        kn = valid_f * (s < 0.5).astype(jnp.float32)
        return (kn, jnp.any(kn != k))

    keep_f, _ = lax.while_loop(cond_fn, body_fn, (valid_f, True))

    sc_ref[0] = sc_top
    bx_ref[0] = bx_top
    keep_ref[0] = keep_f


def _final_kernel(s_row_ref, s_col_ref, k_row_ref, k_col_ref, bx_ref,
                  boxes_out, scores_out, labels_out):
    s_row = s_row_ref[...]
    k_row = k_row_ref[...]
    a_row = jnp.where(k_row > 0.5, s_row, _NEG)
    iidx = lax.broadcasted_iota(jnp.int32, (1, _M), 1)

    def rank_chunk(j, rank):
        s_c = s_col_ref[pl.ds(j * _CH2, _CH2), :]
        k_c = k_col_ref[pl.ds(j * _CH2, _CH2), :]
        a_c = jnp.where(k_c > 0.5, s_c, _NEG)
        jidx = j * _CH2 + lax.broadcasted_iota(jnp.int32, (_CH2, 1), 0)
        cmp = (a_c > a_row) | ((a_c == a_row) &
                               ((s_c > s_row) | ((s_c == s_row) & (jidx < iidx))))
        return rank + jnp.sum(cmp.astype(jnp.float32), axis=0, keepdims=True)

    rank = lax.fori_loop(0, _M // _CH2, rank_chunk,
                         jnp.zeros((1, _M), jnp.float32))

    r_io = lax.broadcasted_iota(jnp.int32, (_DET, _M), 0)
    sel = (r_io == rank.astype(jnp.int32)).astype(jnp.float32)
    boxes_out[...] = lax.dot_general(sel, bx_ref[...], (((1,), (0,)), ((), ())),
                                     preferred_element_type=jnp.float32, precision=lax.Precision.HIGHEST)
    ssel = lax.dot_general(s_row, sel, (((1,), (1,)), ((), ())),
                           preferred_element_type=jnp.float32, precision=lax.Precision.HIGHEST)
    scores_out[...] = jnp.where(ssel <= -1e29, -jnp.inf, ssel)
    lab_row = (iidx // _K + 1).astype(jnp.float32)
    lsel = lax.dot_general(lab_row, sel, (((1,), (1,)), ((), ())),
                           preferred_element_type=jnp.float32, precision=lax.Precision.HIGHEST)
    labels_out[...] = (lsel + 0.5).astype(jnp.int32)


def kernel(bbox_regression, cls_logits, anchors):
    f32 = jnp.float32
    fgc, fgr, boxes = pl.pallas_call(
        _prep_kernel,
        out_shape=[
            jax.ShapeDtypeStruct((_N, _FG), f32),
            jax.ShapeDtypeStruct((_FG, 1, _N), f32),
            jax.ShapeDtypeStruct((_N, 4), f32),
        ],
    )(bbox_regression, cls_logits, anchors)

    sc_top, bx_top, keep = pl.pallas_call(
        _class_kernel,
        grid=(_FG,),
        in_specs=[
            pl.BlockSpec((_N, _FG), lambda c: (0, 0)),
            pl.BlockSpec((1, 1, _N), lambda c: (c, 0, 0)),
            pl.BlockSpec((_N, 4), lambda c: (0, 0)),
        ],
        out_specs=[
            pl.BlockSpec((1, 1, _K), lambda c: (c, 0, 0)),
            pl.BlockSpec((1, _K, 4), lambda c: (c, 0, 0)),
            pl.BlockSpec((1, 1, _K), lambda c: (c, 0, 0)),
        ],
        out_shape=[
            jax.ShapeDtypeStruct((_FG, 1, _K), f32),
            jax.ShapeDtypeStruct((_FG, _K, 4), f32),
            jax.ShapeDtypeStruct((_FG, 1, _K), f32),
        ],
    )(fgc, fgr, boxes)

    s_row = sc_top.reshape(1, _M)
    s_col = sc_top.reshape(_M, 1)
    k_row = keep.reshape(1, _M)
    k_col = keep.reshape(_M, 1)
    bx = bx_top.reshape(_M, 4)

    boxes_o, scores_o, labels_o = pl.pallas_call(
        _final_kernel,
        out_shape=[
            jax.ShapeDtypeStruct((_DET, 4), f32),
            jax.ShapeDtypeStruct((1, _DET), f32),
            jax.ShapeDtypeStruct((1, _DET), jnp.int32),
        ],
    )(s_row, s_col, k_row, k_col, bx)

    return boxes_o, scores_o.reshape(_DET), labels_o.reshape(_DET)
